# baseline (speedup 1.0000x reference)
import functools

import jax
import jax.numpy as jnp
import numpy as np
from jax import lax
from jax.experimental import pallas as pl
from jax.experimental.pallas import tpu as pltpu
from jax.experimental.pallas import tpu_sc as plsc

B, N, D = 1024, 32, 64
NW = 32
BPW = B // NW
P = N * (N - 1) // 2
NQ = D // 16
RB = 8
NT = N // RB


def _pflat(i, j):
    return i * (N - 1) - (i * (i - 1)) // 2 + (j - i - 1)


def _build_order():
    order = []
    for t in range(NT):
        i0 = t * RB
        for r in range(RB):
            for s in range(r + 1, RB):
                order.append(_pflat(i0 + r, i0 + s))
    for t in range(NT):
        i0 = t * RB
        for j0 in range(i0 + RB, N, 2):
            for j in (j0, j0 + 1):
                for r in range(RB):
                    order.append(_pflat(i0 + r, j))
    assert len(order) == P and sorted(order) == list(range(P))
    return np.asarray(order, dtype=np.int32)


_ORDER = _build_order()


def _load_row(emb_v, b, i):
    return [
        emb_v[b, i >> 1, pl.ds((i & 1) * D + q * 16, 16)] for q in range(NQ)
    ]


def _dot(ra, rb):
    m = [ra[q] * rb[q] for q in range(NQ)]
    return jnp.sum((m[0] + m[1]) + (m[2] + m[3]))


def _edge_body(emb_hbm, tab_hbm, out_hbm, emb_v, tab_v, out_v):
    wid = lax.axis_index("s") * 2 + lax.axis_index("c")
    base = wid * BPW
    pltpu.sync_copy(emb_hbm.at[pl.ds(base, BPW)], emb_v)
    pltpu.sync_copy(tab_hbm, tab_v)
    lane = lax.iota(jnp.int32, 16)
    zeros = jnp.zeros((16,), jnp.float32)

    @plsc.parallel_loop(0, BPW)
    def batch_body(b):
        bvec = jnp.full((16,), b, jnp.int32)

        def flush(coll, g16):
            idx = tab_v[pl.ds(g16, 16)]
            plsc.store_scatter(out_v, [bvec, idx], coll)

        c = 0
        coll = zeros
        for t in range(NT):
            i0 = t * RB
            rows = [_load_row(emb_v, b, i0 + r) for r in range(RB)]
            for r in range(RB):
                for s in range(r + 1, RB):
                    coll = jnp.where(lane == (c & 15), _dot(rows[r], rows[s]), coll)
                    if (c & 15) == 15:
                        flush(coll, (c >> 4) * 16)
                        coll = zeros
                    c += 1
        assert c % 16 == 0
        for t in range(NT - 1):
            i0 = t * RB
            rows = [_load_row(emb_v, b, i0 + r) for r in range(RB)]
            nj = N - (i0 + RB)
            cb = c

            @plsc.parallel_loop(0, nj // 2, unroll=2)
            def m_body(m, rows=rows, cb=cb, bvec=bvec):
                jrow = (i0 + RB) // 2 + m
                rj = [emb_v[b, jrow, pl.ds(q * 16, 16)] for q in range(2 * NQ)]
                coll = zeros
                for u in range(16):
                    dj, r = u // 8, u % 8
                    tot = _dot(rows[r], rj[dj * NQ:(dj + 1) * NQ])
                    coll = jnp.where(lane == u, tot, coll)
                flush(coll, cb + 16 * m)
            c += 8 * nj

    pltpu.sync_copy(out_v, out_hbm.at[pl.ds(base, BPW)])


_edge_kernel = functools.partial(
    pl.kernel,
    out_type=jax.ShapeDtypeStruct((B, P), jnp.float32),
    mesh=plsc.VectorSubcoreMesh(core_axis_name="c", subcore_axis_name="s"),
    scratch_types=[
        pltpu.VMEM((BPW, N // 2, 2 * D), jnp.float32),
        pltpu.VMEM((P,), jnp.int32),
        pltpu.VMEM((BPW, P), jnp.float32),
    ],
    compiler_params=pltpu.CompilerParams(needs_layout_passes=False),
)(_edge_body)


def kernel(embeddings):
    i, j = jnp.triu_indices(N, k=1)
    node_combinations = jnp.stack([i, j], axis=1)
    edge_logits = _edge_kernel(
        embeddings.reshape(B, N // 2, 2 * D), jnp.asarray(_ORDER)
    )
    return (edge_logits, node_combinations)


# speedup vs baseline: 1.1235x; 1.1235x over previous
import functools

import jax
import jax.numpy as jnp
import numpy as np
from jax import lax
from jax.experimental import pallas as pl
from jax.experimental.pallas import tpu as pltpu
from jax.experimental.pallas import tpu_sc as plsc

B, N, D = 1024, 32, 64
NW = 32
BPW = B // NW
P = N * (N - 1) // 2
NQ = D // 16
RB = 8
NT = N // RB


def _pflat(i, j):
    return i * (N - 1) - (i * (i - 1)) // 2 + (j - i - 1)


def _build_order():
    order = []
    for t in range(NT):
        i0 = t * RB
        for r in range(RB):
            for s in range(r + 1, RB):
                order.append(_pflat(i0 + r, i0 + s))
    for t in range(NT):
        i0 = t * RB
        for j0 in range(i0 + RB, N, 2):
            for j in (j0, j0 + 1):
                for r in range(RB):
                    order.append(_pflat(i0 + r, j))
    assert len(order) == P and sorted(order) == list(range(P))
    return np.asarray(order, dtype=np.int32)


_ORDER = _build_order()


def _load_row(emb_v, b, i):
    return [
        emb_v[b, i >> 1, pl.ds((i & 1) * D + q * 16, 16)] for q in range(NQ)
    ]


def _dot(ra, rb):
    m = [ra[q] * rb[q] for q in range(NQ)]
    return jnp.sum((m[0] + m[1]) + (m[2] + m[3]))


def _edge_body(emb_hbm, tab_hbm, out_hbm, emb_v, tab_v, out_v):
    wid = lax.axis_index("s") * 2 + lax.axis_index("c")
    base = wid * BPW
    pltpu.sync_copy(emb_hbm.at[pl.ds(base, BPW)], emb_v)
    pltpu.sync_copy(tab_hbm, tab_v)
    lane = lax.iota(jnp.int32, 16)
    zeros = jnp.zeros((16,), jnp.float32)

    @plsc.parallel_loop(0, BPW)
    def batch_body(b):
        bvec = jnp.full((16,), b, jnp.int32)

        def flush(coll, g16):
            idx = tab_v[pl.ds(g16, 16)]
            plsc.store_scatter(out_v, [bvec, idx], coll)

        c = 0
        coll = zeros
        for t in range(NT):
            i0 = t * RB
            rows = [_load_row(emb_v, b, i0 + r) for r in range(RB)]
            for r in range(RB):
                for s in range(r + 1, RB):
                    coll = jnp.where(lane == (c & 15), _dot(rows[r], rows[s]), coll)
                    if (c & 15) == 15:
                        flush(coll, (c >> 4) * 16)
                        coll = zeros
                    c += 1
        assert c % 16 == 0
        for t in range(NT - 1):
            i0 = t * RB
            rows = [_load_row(emb_v, b, i0 + r) for r in range(RB)]
            nj = N - (i0 + RB)
            cb = c

            @plsc.parallel_loop(0, nj // 2, unroll=2)
            def m_body(m, rows=rows, cb=cb, bvec=bvec):
                jrow = (i0 + RB) // 2 + m
                rj = [emb_v[b, jrow, pl.ds(q * 16, 16)] for q in range(2 * NQ)]
                coll = zeros
                for u in range(16):
                    dj, r = u // 8, u % 8
                    tot = _dot(rows[r], rj[dj * NQ:(dj + 1) * NQ])
                    coll = jnp.where(lane == u, tot, coll)
                flush(coll, cb + 16 * m)
            c += 8 * nj

    pltpu.sync_copy(out_v, out_hbm.at[pl.ds(base, BPW)])


_edge_kernel = functools.partial(
    pl.kernel,
    out_type=jax.ShapeDtypeStruct((B, P), jnp.float32),
    mesh=plsc.VectorSubcoreMesh(core_axis_name="c", subcore_axis_name="s"),
    scratch_types=[
        pltpu.VMEM((BPW, N // 2, 2 * D), jnp.float32),
        pltpu.VMEM((P,), jnp.int32),
        pltpu.VMEM((BPW, P), jnp.float32),
    ],
    compiler_params=pltpu.CompilerParams(needs_layout_passes=False),
)(_edge_body)


_NODE_COMBINATIONS = np.stack(np.triu_indices(N, k=1), axis=1).astype(np.int32)


def kernel(embeddings):
    node_combinations = jnp.asarray(_NODE_COMBINATIONS)
    edge_logits = _edge_kernel(
        embeddings.reshape(B, N // 2, 2 * D), jnp.asarray(_ORDER)
    )
    return (edge_logits, node_combinations)


# speedup vs baseline: 1.4124x; 1.2572x over previous
import functools

import jax
import jax.numpy as jnp
import numpy as np
from jax import lax
from jax.experimental import pallas as pl
from jax.experimental.pallas import tpu as pltpu
from jax.experimental.pallas import tpu_sc as plsc

B, N, D = 1024, 32, 64
B_SC = 512
NW = 32
BPW = B_SC // NW
P = N * (N - 1) // 2
NQ = D // 16
RB = 8
NT = N // RB


def _pflat(i, j):
    return i * (N - 1) - (i * (i - 1)) // 2 + (j - i - 1)


def _build_order():
    order = []
    for t in range(NT):
        i0 = t * RB
        for r in range(RB):
            for s in range(r + 1, RB):
                order.append(_pflat(i0 + r, i0 + s))
    for t in range(NT):
        i0 = t * RB
        for j0 in range(i0 + RB, N, 2):
            for j in (j0, j0 + 1):
                for r in range(RB):
                    order.append(_pflat(i0 + r, j))
    assert len(order) == P and sorted(order) == list(range(P))
    return np.asarray(order, dtype=np.int32)


_ORDER = _build_order()


def _load_row(emb_v, b, i):
    return [
        emb_v[b, i >> 1, pl.ds((i & 1) * D + q * 16, 16)] for q in range(NQ)
    ]


def _dot(ra, rb):
    m = [ra[q] * rb[q] for q in range(NQ)]
    return jnp.sum((m[0] + m[1]) + (m[2] + m[3]))


def _edge_body(emb_hbm, tab_hbm, out_hbm, emb_v, tab_v, out_v):
    wid = lax.axis_index("s") * 2 + lax.axis_index("c")
    base = wid * BPW
    pltpu.sync_copy(emb_hbm.at[pl.ds(base, BPW)], emb_v)
    pltpu.sync_copy(tab_hbm, tab_v)
    lane = lax.iota(jnp.int32, 16)
    zeros = jnp.zeros((16,), jnp.float32)

    @plsc.parallel_loop(0, BPW)
    def batch_body(b):
        bvec = jnp.full((16,), b, jnp.int32)

        def flush(coll, g16):
            idx = tab_v[pl.ds(g16, 16)]
            plsc.store_scatter(out_v, [bvec, idx], coll)

        c = 0
        coll = zeros
        for t in range(NT):
            i0 = t * RB
            rows = [_load_row(emb_v, b, i0 + r) for r in range(RB)]
            for r in range(RB):
                for s in range(r + 1, RB):
                    coll = jnp.where(lane == (c & 15), _dot(rows[r], rows[s]), coll)
                    if (c & 15) == 15:
                        flush(coll, (c >> 4) * 16)
                        coll = zeros
                    c += 1
        assert c % 16 == 0
        for t in range(NT - 1):
            i0 = t * RB
            rows = [_load_row(emb_v, b, i0 + r) for r in range(RB)]
            nj = N - (i0 + RB)
            cb = c

            @plsc.parallel_loop(0, nj // 2, unroll=2)
            def m_body(m, rows=rows, cb=cb, bvec=bvec):
                jrow = (i0 + RB) // 2 + m
                rj = [emb_v[b, jrow, pl.ds(q * 16, 16)] for q in range(2 * NQ)]
                coll = zeros
                for u in range(16):
                    dj, r = u // 8, u % 8
                    tot = _dot(rows[r], rj[dj * NQ:(dj + 1) * NQ])
                    coll = jnp.where(lane == u, tot, coll)
                flush(coll, cb + 16 * m)
            c += 8 * nj

    pltpu.sync_copy(out_v, out_hbm.at[pl.ds(base, BPW)])


_edge_kernel = functools.partial(
    pl.kernel,
    out_type=jax.ShapeDtypeStruct((B_SC, P), jnp.float32),
    mesh=plsc.VectorSubcoreMesh(core_axis_name="c", subcore_axis_name="s"),
    scratch_types=[
        pltpu.VMEM((BPW, N // 2, 2 * D), jnp.float32),
        pltpu.VMEM((P,), jnp.int32),
        pltpu.VMEM((BPW, P), jnp.float32),
    ],
    compiler_params=pltpu.CompilerParams(needs_layout_passes=False),
)(_edge_body)


_NODE_COMBINATIONS = np.stack(np.triu_indices(N, k=1), axis=1).astype(np.int32)

BB = 128


def _tc_body(emb_ref, out_ref):
    e = emb_ref[...]
    g = lax.dot_general(
        e, e,
        dimension_numbers=(((2,), (2,)), ((0,), (0,))),
        preferred_element_type=jnp.float32,
    )
    out_ref[...] = jnp.concatenate(
        [g[:, i, i + 1:] for i in range(N - 1)], axis=-1
    )


_tc_kernel = pl.pallas_call(
    _tc_body,
    grid=((B - B_SC) // BB,),
    in_specs=[pl.BlockSpec((BB, N, D), lambda m: (m + B_SC // BB, 0, 0))],
    out_specs=pl.BlockSpec((BB, P), lambda m: (m, 0)),
    out_shape=jax.ShapeDtypeStruct((B - B_SC, P), jnp.float32),
)


def kernel(embeddings):
    node_combinations = jnp.asarray(_NODE_COMBINATIONS)
    sc_out = _edge_kernel(
        embeddings.reshape(B, N // 2, 2 * D), jnp.asarray(_ORDER)
    )
    tc_out = _tc_kernel(embeddings)
    edge_logits = jnp.concatenate([sc_out, tc_out], axis=0)
    return (edge_logits, node_combinations)


# speedup vs baseline: 1.4623x; 1.0353x over previous
import functools

import jax
import jax.numpy as jnp
import numpy as np
from jax import lax
from jax.experimental import pallas as pl
from jax.experimental.pallas import tpu as pltpu
from jax.experimental.pallas import tpu_sc as plsc

B, N, D = 1024, 32, 64
B_SC = 256
NW = 16
BPW = B_SC // NW
P = N * (N - 1) // 2
NQ = D // 16
RB = 8
NT = N // RB


def _pflat(i, j):
    return i * (N - 1) - (i * (i - 1)) // 2 + (j - i - 1)


def _build_order():
    order = []
    for t in range(NT):
        i0 = t * RB
        for r in range(RB):
            for s in range(r + 1, RB):
                order.append(_pflat(i0 + r, i0 + s))
    for t in range(NT):
        i0 = t * RB
        for j0 in range(i0 + RB, N, 2):
            for j in (j0, j0 + 1):
                for r in range(RB):
                    order.append(_pflat(i0 + r, j))
    assert len(order) == P and sorted(order) == list(range(P))
    return np.asarray(order, dtype=np.int32)


_ORDER = _build_order()


def _load_row(emb_v, b, i):
    return [
        emb_v[b, i >> 1, pl.ds((i & 1) * D + q * 16, 16)] for q in range(NQ)
    ]


def _dot(ra, rb):
    m = [ra[q] * rb[q] for q in range(NQ)]
    return jnp.sum((m[0] + m[1]) + (m[2] + m[3]))


def _edge_body(emb_hbm, tab_hbm, out_hbm, emb_v, tab_v, out_v):
    wid = lax.axis_index("s")
    base = wid * BPW
    pltpu.sync_copy(emb_hbm.at[pl.ds(base, BPW)], emb_v)
    pltpu.sync_copy(tab_hbm, tab_v)
    lane = lax.iota(jnp.int32, 16)
    zeros = jnp.zeros((16,), jnp.float32)

    @plsc.parallel_loop(0, BPW)
    def batch_body(b):
        bvec = jnp.full((16,), b, jnp.int32)

        def flush(coll, g16):
            idx = tab_v[pl.ds(g16, 16)]
            plsc.store_scatter(out_v, [bvec, idx], coll)

        c = 0
        coll = zeros
        for t in range(NT):
            i0 = t * RB
            rows = [_load_row(emb_v, b, i0 + r) for r in range(RB)]
            for r in range(RB):
                for s in range(r + 1, RB):
                    coll = jnp.where(lane == (c & 15), _dot(rows[r], rows[s]), coll)
                    if (c & 15) == 15:
                        flush(coll, (c >> 4) * 16)
                        coll = zeros
                    c += 1
        assert c % 16 == 0
        for t in range(NT - 1):
            i0 = t * RB
            rows = [_load_row(emb_v, b, i0 + r) for r in range(RB)]
            nj = N - (i0 + RB)
            cb = c

            @plsc.parallel_loop(0, nj // 2, unroll=2)
            def m_body(m, rows=rows, cb=cb, bvec=bvec):
                jrow = (i0 + RB) // 2 + m
                rj = [emb_v[b, jrow, pl.ds(q * 16, 16)] for q in range(2 * NQ)]
                coll = zeros
                for u in range(16):
                    dj, r = u // 8, u % 8
                    tot = _dot(rows[r], rj[dj * NQ:(dj + 1) * NQ])
                    coll = jnp.where(lane == u, tot, coll)
                flush(coll, cb + 16 * m)
            c += 8 * nj

    pltpu.sync_copy(out_v, out_hbm.at[pl.ds(base, BPW)])


_edge_kernel = functools.partial(
    pl.kernel,
    out_type=jax.ShapeDtypeStruct((B_SC, P), jnp.float32),
    mesh=plsc.VectorSubcoreMesh(core_axis_name="c", subcore_axis_name="s", num_cores=1),
    scratch_types=[
        pltpu.VMEM((BPW, N // 2, 2 * D), jnp.float32),
        pltpu.VMEM((P,), jnp.int32),
        pltpu.VMEM((BPW, P), jnp.float32),
    ],
    compiler_params=pltpu.CompilerParams(needs_layout_passes=False),
)(_edge_body)


_NODE_COMBINATIONS = np.stack(np.triu_indices(N, k=1), axis=1).astype(np.int32)

BB = 128


def _tc_body(emb_ref, out_ref):
    e = emb_ref[...]
    g = lax.dot_general(
        e, e,
        dimension_numbers=(((2,), (2,)), ((0,), (0,))),
        preferred_element_type=jnp.float32,
    )
    out_ref[...] = jnp.concatenate(
        [g[:, i, i + 1:] for i in range(N - 1)], axis=-1
    )


_tc_kernel = pl.pallas_call(
    _tc_body,
    grid=((B - B_SC) // BB,),
    in_specs=[pl.BlockSpec((BB, N, D), lambda m: (m + B_SC // BB, 0, 0))],
    out_specs=pl.BlockSpec((BB, P), lambda m: (m, 0)),
    out_shape=jax.ShapeDtypeStruct((B - B_SC, P), jnp.float32),
)


def kernel(embeddings):
    node_combinations = jnp.asarray(_NODE_COMBINATIONS)
    sc_out = _edge_kernel(
        embeddings.reshape(B, N // 2, 2 * D), jnp.asarray(_ORDER)
    )
    tc_out = _tc_kernel(embeddings)
    edge_logits = jnp.concatenate([sc_out, tc_out], axis=0)
    return (edge_logits, node_combinations)
